# baseline (speedup 1.0000x reference)
import functools

import jax
import jax.numpy as jnp
from jax import lax
from jax.experimental import pallas as pl
from jax.experimental.pallas import tpu as pltpu
from jax.experimental.pallas import tpu_sc as plsc

N = 10000
D = 128
E = 320000
NPAD = 10240
CHUNK = 128
NC, NS = 2, 16
NWORK = NC * NS
NSTEP = 80
EPW = NSTEP * CHUNK
EPADTOT = EPW * NWORK
RPT = NPAD // NS
GB = 8


def _zero_stripe(sid, zbuf, acc):
    for t in range(RPT // CHUNK):
        pltpu.sync_copy(zbuf, acc.at[pl.ds(sid * RPT + t * CHUNK, CHUNK)])


def _dump_stripe(cid, sid, acc, bounce, out):
    for t in range(RPT // CHUNK):
        off = sid * RPT + t * CHUNK
        pltpu.sync_copy(acc.at[pl.ds(off, CHUNK)], bounce)
        pltpu.sync_copy(bounce, out.at[cid, pl.ds(off, CHUNK)])


def _make_agg():
    mesh = plsc.VectorSubcoreMesh(core_axis_name="c", subcore_axis_name="s")
    out_type = jax.ShapeDtypeStruct((NC, NPAD, D), jnp.float32)
    scratch = [
        pltpu.VMEM((GB, CHUNK), jnp.int32),
        pltpu.VMEM((GB, CHUNK), jnp.int32),
        pltpu.VMEM((CHUNK, D), jnp.float32),
        pltpu.VMEM((CHUNK, D), jnp.float32),
        pltpu.VMEM_SHARED((NPAD, D), jnp.float32),
        pltpu.SemaphoreType.DMA,
        pltpu.SemaphoreType.DMA,
        pltpu.SemaphoreType.DMA,
        pltpu.SemaphoreType.DMA,
    ]

    def body(table, src_i, dst_i, zf, out_sum, s_v, d_v, buf0, buf1, acc,
             gs0, gs1, ss0, ss1):
        cid = lax.axis_index("c")
        sid = lax.axis_index("s")
        w = cid * NS + sid
        pltpu.sync_copy(zf, buf0)
        _zero_stripe(sid, buf0, acc)
        plsc.subcore_barrier()

        bufs = (buf0, buf1)
        gsems = (gs0, gs1)
        ssems = (ss0, ss1)

        def group(g, carry):
            base = pl.multiple_of(w * NSTEP + g * GB, GB)
            pltpu.sync_copy(src_i.at[pl.ds(base, GB)], s_v)
            pltpu.sync_copy(dst_i.at[pl.ds(base, GB)], d_v)
            gcp = [None, None]
            scp = [None, None]
            gcp[0] = pltpu.async_copy(table.at[s_v.at[0]], buf0, gs0)
            for j in range(GB):
                b = j % 2
                gcp[b].wait()
                if j + 1 < GB:
                    nb = (j + 1) % 2
                    if scp[nb] is not None:
                        scp[nb].wait()
                    gcp[nb] = pltpu.async_copy(table.at[s_v.at[j + 1]],
                                               bufs[nb], gsems[nb])
                scp[b] = pltpu.async_copy(bufs[b], acc.at[d_v.at[j]],
                                          ssems[b], add=True)
            scp[0].wait()
            scp[1].wait()
            return carry

        lax.fori_loop(0, NSTEP // GB, group, 0)
        plsc.subcore_barrier()
        _dump_stripe(cid, sid, acc, buf0, out_sum)

    return pl.kernel(body, out_type=out_type, mesh=mesh,
                     scratch_types=scratch)


def _make_cnt():
    mesh = plsc.VectorSubcoreMesh(core_axis_name="c", subcore_axis_name="s")
    out_type = jax.ShapeDtypeStruct((NC, NPAD, D), jnp.float32)
    scratch = [
        pltpu.VMEM((GB, CHUNK), jnp.int32),
        pltpu.VMEM((CHUNK, D), jnp.float32),
        pltpu.VMEM_SHARED((NPAD, D), jnp.float32),
        pltpu.SemaphoreType.DMA,
    ]

    def body(dst_i, zf, ones_h, out_cnt, d_v, obuf, acc, sem):
        cid = lax.axis_index("c")
        sid = lax.axis_index("s")
        w = cid * NS + sid
        pltpu.sync_copy(zf, obuf)
        _zero_stripe(sid, obuf, acc)
        pltpu.sync_copy(ones_h, obuf)
        plsc.subcore_barrier()

        def group(g, carry):
            base = pl.multiple_of(w * NSTEP + g * GB, GB)
            pltpu.sync_copy(dst_i.at[pl.ds(base, GB)], d_v)
            cps = [pltpu.async_copy(obuf, acc.at[d_v.at[j]], sem, add=True)
                   for j in range(GB)]
            for cp in cps:
                cp.wait()
            return carry

        lax.fori_loop(0, NSTEP // GB, group, 0)
        plsc.subcore_barrier()
        _dump_stripe(cid, sid, acc, obuf, out_cnt)

    return pl.kernel(body, out_type=out_type, mesh=mesh,
                     scratch_types=scratch)


_AGG = _make_agg()
_CNT = _make_cnt()


def _dense_body(n_rel, relu, *refs):
    x_ref = refs[0]
    o_ref = refs[-1]
    rel_refs = refs[1:-1]
    x = x_ref[...]
    acc = None
    wr_sum = None
    bias = None
    for r in range(n_rel):
        p_ref, c_ref, wl_ref, bl_ref, wr_ref = rel_refs[5 * r:5 * r + 5]
        p = p_ref[0] + p_ref[1]
        c = c_ref[0, :, 0:1].astype(jnp.float32) + \
            c_ref[1, :, 0:1].astype(jnp.float32)
        a = p / jnp.maximum(c, 1.0)
        t = lax.dot_general(a, wl_ref[...], (((1,), (1,)), ((), ())),
                            preferred_element_type=jnp.float32)
        acc = t if acc is None else acc + t
        wr = wr_ref[...]
        wr_sum = wr if wr_sum is None else wr_sum + wr
        b = bl_ref[...]
        bias = b if bias is None else bias + b
    acc = acc + lax.dot_general(x, wr_sum, (((1,), (1,)), ((), ())),
                                preferred_element_type=jnp.float32)
    acc = acc + bias
    if relu:
        acc = jnp.maximum(acc, 0.0)
    o_ref[...] = acc


def _dense(x_dst, rels, relu):
    n_rel = len(rels)
    blk = 1000
    grid = (N // blk,)
    in_specs = [pl.BlockSpec((blk, D), lambda i: (i, 0))]
    args = [x_dst]
    for (p, c, wl, bl, wr) in rels:
        in_specs += [
            pl.BlockSpec((NC, blk, D), lambda i: (0, i, 0)),
            pl.BlockSpec((NC, blk, 1), lambda i: (0, i, 0)),
            pl.BlockSpec((D, D), lambda i: (0, 0)),
            pl.BlockSpec((1, D), lambda i: (0, 0)),
            pl.BlockSpec((D, D), lambda i: (0, 0)),
        ]
        args += [p, c, wl, bl.reshape(1, D), wr]
    return pl.pallas_call(
        functools.partial(_dense_body, n_rel, relu),
        grid=grid,
        in_specs=in_specs,
        out_specs=pl.BlockSpec((blk, D), lambda i: (i, 0)),
        out_shape=jax.ShapeDtypeStruct((N, D), jnp.float32),
    )(*args)


def _prep_edges(ei):
    src = ei[0].astype(jnp.int32)
    dst = ei[1].astype(jnp.int32)
    pad = EPADTOT - E
    src = jnp.concatenate([src, jnp.zeros((pad,), jnp.int32)])
    dst = jnp.concatenate([dst, jnp.full((pad,), N, jnp.int32)])
    return (src.reshape(EPADTOT // CHUNK, CHUNK),
            dst.reshape(EPADTOT // CHUNK, CHUNK))


def kernel(x_user, x_item, edge_index_reviews, edge_index_rev_reviews,
           edge_index_also_bought, l1_rev_Wl, l1_rev_bl, l1_rev_Wr,
           l1_rr_Wl, l1_rr_bl, l1_rr_Wr, l1_ab_Wl, l1_ab_bl, l1_ab_Wr,
           l2_rev_Wl, l2_rev_bl, l2_rev_Wr, l2_rr_Wl, l2_rr_bl, l2_rr_Wr,
           l2_ab_Wl, l2_ab_bl, l2_ab_Wr):
    s_rev, d_rev = _prep_edges(edge_index_reviews)
    s_rr, d_rr = _prep_edges(edge_index_rev_reviews)
    s_ab, d_ab = _prep_edges(edge_index_also_bought)
    zf = jnp.zeros((CHUNK, D), jnp.float32)
    ones_h = jnp.ones((CHUNK, D), jnp.float32)

    c_rev = _CNT(d_rev, zf, ones_h)[:, :, 0:1]
    c_ab = _CNT(d_ab, zf, ones_h)[:, :, 0:1]
    c_rr = _CNT(d_rr, zf, ones_h)[:, :, 0:1]

    p_rev = _AGG(x_user, s_rev, d_rev, zf)
    p_ab = _AGG(x_item, s_ab, d_ab, zf)
    p_rr = _AGG(x_item, s_rr, d_rr, zf)

    item1 = _dense(x_item, [(p_rev, c_rev, l1_rev_Wl, l1_rev_bl, l1_rev_Wr),
                            (p_ab, c_ab, l1_ab_Wl, l1_ab_bl, l1_ab_Wr)],
                   relu=True)
    user1 = _dense(x_user, [(p_rr, c_rr, l1_rr_Wl, l1_rr_bl, l1_rr_Wr)],
                   relu=True)

    q_rev = _AGG(user1, s_rev, d_rev, zf)
    q_ab = _AGG(item1, s_ab, d_ab, zf)
    q_rr = _AGG(item1, s_rr, d_rr, zf)

    item2 = _dense(item1, [(q_rev, c_rev, l2_rev_Wl, l2_rev_bl, l2_rev_Wr),
                           (q_ab, c_ab, l2_ab_Wl, l2_ab_bl, l2_ab_Wr)],
                   relu=False)
    user2 = _dense(user1, [(q_rr, c_rr, l2_rr_Wl, l2_rr_bl, l2_rr_Wr)],
                   relu=False)
    return (user2, item2)


# speedup vs baseline: 1.0138x; 1.0138x over previous
import functools

import jax
import jax.numpy as jnp
from jax import lax
from jax.experimental import pallas as pl
from jax.experimental.pallas import tpu as pltpu
from jax.experimental.pallas import tpu_sc as plsc

N = 10000
D = 128
E = 320000
NPAD = 10240
CHUNK = 128
NC, NS = 2, 16
NWORK = NC * NS
NSTEP = 80
EPW = NSTEP * CHUNK
EPADTOT = EPW * NWORK
RPT = NPAD // NS
GB = 16


def _zero_stripe(sid, zbuf, acc):
    for t in range(RPT // CHUNK):
        pltpu.sync_copy(zbuf, acc.at[pl.ds(sid * RPT + t * CHUNK, CHUNK)])


def _dump_stripe(cid, sid, acc, bounce, out):
    for t in range(RPT // CHUNK):
        off = sid * RPT + t * CHUNK
        pltpu.sync_copy(acc.at[pl.ds(off, CHUNK)], bounce)
        pltpu.sync_copy(bounce, out.at[cid, pl.ds(off, CHUNK)])


def _make_agg():
    mesh = plsc.VectorSubcoreMesh(core_axis_name="c", subcore_axis_name="s")
    out_type = jax.ShapeDtypeStruct((NC, NPAD, D), jnp.float32)
    scratch = [
        pltpu.VMEM((GB, CHUNK), jnp.int32),
        pltpu.VMEM((GB, CHUNK), jnp.int32),
        pltpu.VMEM((CHUNK, D), jnp.float32),
        pltpu.VMEM((CHUNK, D), jnp.float32),
        pltpu.VMEM_SHARED((NPAD, D), jnp.float32),
        pltpu.SemaphoreType.DMA,
        pltpu.SemaphoreType.DMA,
        pltpu.SemaphoreType.DMA,
        pltpu.SemaphoreType.DMA,
    ]

    def body(table, src_i, dst_i, zf, out_sum, s_v, d_v, buf0, buf1, acc,
             gs0, gs1, ss0, ss1):
        cid = lax.axis_index("c")
        sid = lax.axis_index("s")
        w = cid * NS + sid
        pltpu.sync_copy(zf, buf0)
        _zero_stripe(sid, buf0, acc)
        plsc.subcore_barrier()

        bufs = (buf0, buf1)
        gsems = (gs0, gs1)
        ssems = (ss0, ss1)

        def group(g, carry):
            base = pl.multiple_of(w * NSTEP + g * GB, GB)
            pltpu.sync_copy(src_i.at[pl.ds(base, GB)], s_v)
            pltpu.sync_copy(dst_i.at[pl.ds(base, GB)], d_v)
            gcp = [None, None]
            scp = [None, None]
            gcp[0] = pltpu.async_copy(table.at[s_v.at[0]], buf0, gs0)
            for j in range(GB):
                b = j % 2
                gcp[b].wait()
                if j + 1 < GB:
                    nb = (j + 1) % 2
                    if scp[nb] is not None:
                        scp[nb].wait()
                    gcp[nb] = pltpu.async_copy(table.at[s_v.at[j + 1]],
                                               bufs[nb], gsems[nb])
                scp[b] = pltpu.async_copy(bufs[b], acc.at[d_v.at[j]],
                                          ssems[b], add=True)
            scp[0].wait()
            scp[1].wait()
            return carry

        lax.fori_loop(0, NSTEP // GB, group, 0)
        plsc.subcore_barrier()
        _dump_stripe(cid, sid, acc, buf0, out_sum)

    return pl.kernel(body, out_type=out_type, mesh=mesh,
                     scratch_types=scratch)


def _make_cnt():
    mesh = plsc.VectorSubcoreMesh(core_axis_name="c", subcore_axis_name="s")
    out_type = jax.ShapeDtypeStruct((NC, NPAD, D), jnp.float32)
    scratch = [
        pltpu.VMEM((GB, CHUNK), jnp.int32),
        pltpu.VMEM((CHUNK, D), jnp.float32),
        pltpu.VMEM_SHARED((NPAD, D), jnp.float32),
        pltpu.SemaphoreType.DMA,
    ]

    def body(dst_i, zf, ones_h, out_cnt, d_v, obuf, acc, sem):
        cid = lax.axis_index("c")
        sid = lax.axis_index("s")
        w = cid * NS + sid
        pltpu.sync_copy(zf, obuf)
        _zero_stripe(sid, obuf, acc)
        pltpu.sync_copy(ones_h, obuf)
        plsc.subcore_barrier()

        def group(g, carry):
            base = pl.multiple_of(w * NSTEP + g * GB, GB)
            pltpu.sync_copy(dst_i.at[pl.ds(base, GB)], d_v)
            cps = [pltpu.async_copy(obuf, acc.at[d_v.at[j]], sem, add=True)
                   for j in range(GB)]
            for cp in cps:
                cp.wait()
            return carry

        lax.fori_loop(0, NSTEP // GB, group, 0)
        plsc.subcore_barrier()
        _dump_stripe(cid, sid, acc, obuf, out_cnt)

    return pl.kernel(body, out_type=out_type, mesh=mesh,
                     scratch_types=scratch)


_AGG = _make_agg()
_CNT = _make_cnt()


def _dense_body(n_rel, relu, *refs):
    x_ref = refs[0]
    o_ref = refs[-1]
    rel_refs = refs[1:-1]
    x = x_ref[...]
    acc = None
    wr_sum = None
    bias = None
    for r in range(n_rel):
        p_ref, c_ref, wl_ref, bl_ref, wr_ref = rel_refs[5 * r:5 * r + 5]
        p = p_ref[0] + p_ref[1]
        c = c_ref[0, :, 0:1].astype(jnp.float32) + \
            c_ref[1, :, 0:1].astype(jnp.float32)
        a = p / jnp.maximum(c, 1.0)
        t = lax.dot_general(a, wl_ref[...], (((1,), (1,)), ((), ())),
                            preferred_element_type=jnp.float32)
        acc = t if acc is None else acc + t
        wr = wr_ref[...]
        wr_sum = wr if wr_sum is None else wr_sum + wr
        b = bl_ref[...]
        bias = b if bias is None else bias + b
    acc = acc + lax.dot_general(x, wr_sum, (((1,), (1,)), ((), ())),
                                preferred_element_type=jnp.float32)
    acc = acc + bias
    if relu:
        acc = jnp.maximum(acc, 0.0)
    o_ref[...] = acc


def _dense(x_dst, rels, relu):
    n_rel = len(rels)
    blk = 1000
    grid = (N // blk,)
    in_specs = [pl.BlockSpec((blk, D), lambda i: (i, 0))]
    args = [x_dst]
    for (p, c, wl, bl, wr) in rels:
        in_specs += [
            pl.BlockSpec((NC, blk, D), lambda i: (0, i, 0)),
            pl.BlockSpec((NC, blk, 1), lambda i: (0, i, 0)),
            pl.BlockSpec((D, D), lambda i: (0, 0)),
            pl.BlockSpec((1, D), lambda i: (0, 0)),
            pl.BlockSpec((D, D), lambda i: (0, 0)),
        ]
        args += [p, c, wl, bl.reshape(1, D), wr]
    return pl.pallas_call(
        functools.partial(_dense_body, n_rel, relu),
        grid=grid,
        in_specs=in_specs,
        out_specs=pl.BlockSpec((blk, D), lambda i: (i, 0)),
        out_shape=jax.ShapeDtypeStruct((N, D), jnp.float32),
    )(*args)


def _prep_edges(ei):
    src = ei[0].astype(jnp.int32)
    dst = ei[1].astype(jnp.int32)
    pad = EPADTOT - E
    src = jnp.concatenate([src, jnp.zeros((pad,), jnp.int32)])
    dst = jnp.concatenate([dst, jnp.full((pad,), N, jnp.int32)])
    return (src.reshape(EPADTOT // CHUNK, CHUNK),
            dst.reshape(EPADTOT // CHUNK, CHUNK))


def kernel(x_user, x_item, edge_index_reviews, edge_index_rev_reviews,
           edge_index_also_bought, l1_rev_Wl, l1_rev_bl, l1_rev_Wr,
           l1_rr_Wl, l1_rr_bl, l1_rr_Wr, l1_ab_Wl, l1_ab_bl, l1_ab_Wr,
           l2_rev_Wl, l2_rev_bl, l2_rev_Wr, l2_rr_Wl, l2_rr_bl, l2_rr_Wr,
           l2_ab_Wl, l2_ab_bl, l2_ab_Wr):
    s_rev, d_rev = _prep_edges(edge_index_reviews)
    s_rr, d_rr = _prep_edges(edge_index_rev_reviews)
    s_ab, d_ab = _prep_edges(edge_index_also_bought)
    zf = jnp.zeros((CHUNK, D), jnp.float32)
    ones_h = jnp.ones((CHUNK, D), jnp.float32)

    c_rev = _CNT(d_rev, zf, ones_h)[:, :, 0:1]
    c_ab = _CNT(d_ab, zf, ones_h)[:, :, 0:1]
    c_rr = _CNT(d_rr, zf, ones_h)[:, :, 0:1]

    p_rev = _AGG(x_user, s_rev, d_rev, zf)
    p_ab = _AGG(x_item, s_ab, d_ab, zf)
    p_rr = _AGG(x_item, s_rr, d_rr, zf)

    item1 = _dense(x_item, [(p_rev, c_rev, l1_rev_Wl, l1_rev_bl, l1_rev_Wr),
                            (p_ab, c_ab, l1_ab_Wl, l1_ab_bl, l1_ab_Wr)],
                   relu=True)
    user1 = _dense(x_user, [(p_rr, c_rr, l1_rr_Wl, l1_rr_bl, l1_rr_Wr)],
                   relu=True)

    q_rev = _AGG(user1, s_rev, d_rev, zf)
    q_ab = _AGG(item1, s_ab, d_ab, zf)
    q_rr = _AGG(item1, s_rr, d_rr, zf)

    item2 = _dense(item1, [(q_rev, c_rev, l2_rev_Wl, l2_rev_bl, l2_rev_Wr),
                           (q_ab, c_ab, l2_ab_Wl, l2_ab_bl, l2_ab_Wr)],
                   relu=False)
    user2 = _dense(user1, [(q_rr, c_rr, l2_rr_Wl, l2_rr_bl, l2_rr_Wr)],
                   relu=False)
    return (user2, item2)
